# Initial kernel scaffold; baseline (speedup 1.0000x reference)
import jax
import jax.numpy as jnp
from jax.experimental import pallas as pl


def kernel(x, edge_index, batch, Wl0, bl0, Wr0, Wl1, bl1, Wr1, Wl2, bl2, Wr2, Wfc1, bfc1, Wfc2, bfc2):
    raise NotImplementedError("write your pallas kernel here")


# speedup vs baseline: 1.0277x; 1.0277x over previous
import jax
import jax.numpy as jnp
from jax.experimental import pallas as pl

N = 10000
E = 320000
D = 128
H = 128
G = 64
T = 10


def _layer_mm(h, aggm, WlT, bl, WrT):
    n, d = h.shape
    hh = WlT.shape[1]
    BLK = 1000

    def body(h_ref, a_ref, wl_ref, bl_ref, wr_ref, o_ref):
        o_ref[...] = (
            jnp.dot(a_ref[...], wl_ref[...], preferred_element_type=jnp.float32)
            + bl_ref[...]
            + jnp.dot(h_ref[...], wr_ref[...], preferred_element_type=jnp.float32)
        )

    return pl.pallas_call(
        body,
        grid=(n // BLK,),
        in_specs=[
            pl.BlockSpec((BLK, d), lambda i: (i, 0)),
            pl.BlockSpec((BLK, d), lambda i: (i, 0)),
            pl.BlockSpec((d, hh), lambda i: (0, 0)),
            pl.BlockSpec((1, hh), lambda i: (0, 0)),
            pl.BlockSpec((d, hh), lambda i: (0, 0)),
        ],
        out_specs=pl.BlockSpec((BLK, hh), lambda i: (i, 0)),
        out_shape=jax.ShapeDtypeStruct((n, hh), jnp.float32),
    )(h, aggm, WlT, bl.reshape(1, -1), WrT)


def _mlp(pooled, Wfc1T, bfc1, Wfc2T, bfc2):
    def body(p_ref, w1_ref, b1_ref, w2_ref, b2_ref, o_ref):
        h1 = jax.nn.relu(
            jnp.dot(p_ref[...], w1_ref[...], preferred_element_type=jnp.float32)
            + b1_ref[...]
        )
        o_ref[...] = jax.nn.sigmoid(
            jnp.dot(h1, w2_ref[...], preferred_element_type=jnp.float32) + b2_ref[...]
        )

    return pl.pallas_call(
        body,
        out_shape=jax.ShapeDtypeStruct((G, T), jnp.float32),
    )(pooled, Wfc1T, bfc1.reshape(1, -1), Wfc2T, bfc2.reshape(1, -1))


def kernel(x, edge_index, batch, Wl0, bl0, Wr0, Wl1, bl1, Wr1, Wl2, bl2, Wr2,
           Wfc1, bfc1, Wfc2, bfc2):
    src = edge_index[0]
    dst = edge_index[1]
    cnt = jax.ops.segment_sum(jnp.ones((E,), jnp.float32), dst, num_segments=N)
    inv = 1.0 / jnp.maximum(cnt, 1.0)

    h = x
    outs = []
    for Wl, bl, Wr in ((Wl0, bl0, Wr0), (Wl1, bl1, Wr1), (Wl2, bl2, Wr2)):
        agg = jax.ops.segment_sum(jnp.take(h, src, axis=0), dst, num_segments=N)
        aggm = agg * inv[:, None]
        h = _layer_mm(h, aggm, Wl.T, bl, Wr.T)
        outs.append(h)

    cat = jnp.concatenate(outs, axis=1)
    pooled_sum = jax.ops.segment_sum(cat, batch, num_segments=G)
    gcnt = jax.ops.segment_sum(jnp.ones((N,), jnp.float32), batch, num_segments=G)
    pooled = pooled_sum / jnp.maximum(gcnt, 1.0)[:, None]
    return _mlp(pooled, Wfc1.T, bfc1, Wfc2.T, bfc2)


# speedup vs baseline: 4.6253x; 4.5005x over previous
import jax
import jax.numpy as jnp
from jax import lax
from jax.experimental import pallas as pl
from jax.experimental.pallas import tpu as pltpu
from jax.experimental.pallas import tpu_sc as plsc

N = 10000
E = 320000
D = 128
H = 128
G = 64
T = 10

NC = 2
NS = 16
CW = 128
CH = 79
EP = NC * NS * CH * CW
N_PAD = 10112
RPT = N_PAD // NS
BLK = 1000


def _make_sc_agg(do_cnt: bool):
    mesh = plsc.VectorSubcoreMesh(
        core_axis_name="c", subcore_axis_name="s", num_cores=NC,
        num_subcores=NS)
    out_type = [jax.ShapeDtypeStruct((NC, N_PAD, H), jnp.float32)]
    scratch = [
        pltpu.VMEM((CH, CW), jnp.int32),
        pltpu.VMEM((CH, CW), jnp.int32),
        pltpu.VMEM((CW, H), jnp.float32),
        pltpu.VMEM_SHARED((N_PAD, H), jnp.float32),
        pltpu.SemaphoreType.DMA,
    ]
    if do_cnt:
        out_type.append(jax.ShapeDtypeStruct((NC, N_PAD, H), jnp.float32))

    def _fill(ref, value16):
        @pl.loop(0, CW)
        def _(r):
            for kk in range(H // 16):
                ref[r, pl.ds(kk * 16, 16)] = value16

    def _zero_acc_slice(rows_v, acc, s):
        for t in range(4):
            pltpu.sync_copy(rows_v,
                            acc.at[pl.ds(s * RPT + t * CW, CW)])
        pltpu.sync_copy(rows_v.at[pl.ds(0, RPT - 4 * CW)],
                        acc.at[pl.ds(s * RPT + 4 * CW, RPT - 4 * CW)])

    def body(z, pk, *rest):
        if do_cnt:
            agg, cntp, src_v, dst_v, rows_v, acc, sem = rest
        else:
            agg, src_v, dst_v, rows_v, acc, sem = rest
        c = lax.axis_index("c")
        s = lax.axis_index("s")

        pltpu.sync_copy(pk.at[c, s], src_v)

        @pl.loop(0, CH * (CW // 16))
        def _(i):
            j = i // (CW // 16)
            k = (i % (CW // 16)) * 16
            p = src_v[j, pl.ds(k, 16)]
            dst_v[j, pl.ds(k, 16)] = p >> 14
            src_v[j, pl.ds(k, 16)] = p & 16383

        _fill(rows_v, jnp.zeros((16,), jnp.float32))
        _zero_acc_slice(rows_v, acc, s)
        plsc.subcore_barrier()

        @pl.loop(0, CH)
        def _(j):
            pltpu.async_copy(z.at[src_v.at[j]], rows_v, sem).wait()
            pltpu.sync_copy(rows_v, acc.at[dst_v.at[j]], add=True)

        plsc.subcore_barrier()
        pltpu.sync_copy(acc.at[pl.ds(s * RPT, RPT)],
                        agg.at[c, pl.ds(s * RPT, RPT)])

        if do_cnt:
            plsc.subcore_barrier()
            _fill(rows_v, jnp.zeros((16,), jnp.float32))
            _zero_acc_slice(rows_v, acc, s)
            _fill(rows_v, jnp.ones((16,), jnp.float32))
            plsc.subcore_barrier()

            @pl.loop(0, CH)
            def _(j):
                pltpu.sync_copy(rows_v, acc.at[dst_v.at[j]], add=True)

            plsc.subcore_barrier()
            pltpu.sync_copy(acc.at[pl.ds(s * RPT, RPT)],
                            cntp.at[c, pl.ds(s * RPT, RPT)])

    return pl.kernel(body, out_type=out_type, mesh=mesh,
                     scratch_types=scratch)


_sc_agg0 = _make_sc_agg(True)
_sc_agg = _make_sc_agg(False)


def _zkern(h, WlT):
    def body(h_ref, w_ref, o_ref):
        o_ref[...] = jnp.dot(h_ref[...], w_ref[...],
                             preferred_element_type=jnp.float32)

    return pl.pallas_call(
        body,
        grid=(N // BLK,),
        in_specs=[
            pl.BlockSpec((BLK, H), lambda i: (i, 0)),
            pl.BlockSpec((H, H), lambda i: (0, 0)),
        ],
        out_specs=pl.BlockSpec((BLK, H), lambda i: (i, 0)),
        out_shape=jax.ShapeDtypeStruct((N, H), jnp.float32),
    )(h, WlT)


def _prep_inv(cntp):
    def body(c_ref, o_ref):
        cnt = c_ref[0, :, 0:1] + c_ref[1, :, 0:1]
        o_ref[...] = 1.0 / jnp.maximum(cnt, 1.0)

    return pl.pallas_call(
        body,
        out_shape=jax.ShapeDtypeStruct((N_PAD, 1), jnp.float32),
    )(cntp)


def _comb(agg, inv, h, WrT, bl):
    def body(a0_ref, a1_ref, inv_ref, h_ref, wr_ref, b_ref, o_ref):
        asum = a0_ref[0] + a1_ref[0]
        o_ref[...] = (inv_ref[...] * asum + b_ref[...]
                      + jnp.dot(h_ref[...], wr_ref[...],
                                preferred_element_type=jnp.float32))

    return pl.pallas_call(
        body,
        grid=(N // BLK,),
        in_specs=[
            pl.BlockSpec((1, BLK, H), lambda i: (0, i, 0)),
            pl.BlockSpec((1, BLK, H), lambda i: (1, i, 0)),
            pl.BlockSpec((BLK, 1), lambda i: (i, 0)),
            pl.BlockSpec((BLK, H), lambda i: (i, 0)),
            pl.BlockSpec((H, H), lambda i: (0, 0)),
            pl.BlockSpec((1, H), lambda i: (0, 0)),
        ],
        out_specs=pl.BlockSpec((BLK, H), lambda i: (i, 0)),
        out_shape=jax.ShapeDtypeStruct((N, H), jnp.float32),
    )(agg, agg, inv, h, WrT, bl.reshape(1, -1))


def _pool(batchr, h1, h2, h3):
    def body(b_ref, h1_ref, h2_ref, h3_ref, ps_ref, gc_ref):
        i = pl.program_id(0)

        @pl.when(i == 0)
        def _():
            ps_ref[...] = jnp.zeros_like(ps_ref)
            gc_ref[...] = jnp.zeros_like(gc_ref)

        b = b_ref[0]
        S = (lax.broadcasted_iota(jnp.int32, (G, BLK), 0) == b
             ).astype(jnp.float32)
        cat = jnp.concatenate([h1_ref[...], h2_ref[...], h3_ref[...]],
                              axis=1)
        ps_ref[...] += jnp.dot(S, cat, preferred_element_type=jnp.float32)
        gc_ref[...] += jnp.sum(S, axis=1, keepdims=True)

    return pl.pallas_call(
        body,
        grid=(N // BLK,),
        in_specs=[
            pl.BlockSpec((1, 1, BLK), lambda i: (i, 0, 0)),
            pl.BlockSpec((BLK, H), lambda i: (i, 0)),
            pl.BlockSpec((BLK, H), lambda i: (i, 0)),
            pl.BlockSpec((BLK, H), lambda i: (i, 0)),
        ],
        out_specs=[
            pl.BlockSpec((G, 3 * H), lambda i: (0, 0)),
            pl.BlockSpec((G, 1), lambda i: (0, 0)),
        ],
        out_shape=[
            jax.ShapeDtypeStruct((G, 3 * H), jnp.float32),
            jax.ShapeDtypeStruct((G, 1), jnp.float32),
        ],
    )(batchr, h1, h2, h3)


def _mlp(ps, gc, W1T, b1, W2T, b2):
    def body(ps_ref, gc_ref, w1_ref, b1_ref, w2_ref, b2_ref, o_ref):
        pooled = ps_ref[...] / jnp.maximum(gc_ref[...], 1.0)
        hh = jax.nn.relu(jnp.dot(pooled, w1_ref[...],
                                 preferred_element_type=jnp.float32)
                         + b1_ref[...])
        o_ref[...] = jax.nn.sigmoid(
            jnp.dot(hh, w2_ref[...], preferred_element_type=jnp.float32)
            + b2_ref[...])

    return pl.pallas_call(
        body,
        out_shape=jax.ShapeDtypeStruct((G, T), jnp.float32),
    )(ps, gc, W1T, b1.reshape(1, -1), W2T, b2.reshape(1, -1))


def kernel(x, edge_index, batch, Wl0, bl0, Wr0, Wl1, bl1, Wr1, Wl2, bl2, Wr2,
           Wfc1, bfc1, Wfc2, bfc2):
    src = edge_index[0]
    dst = edge_index[1]
    pad = EP - E
    srcp = jnp.concatenate([src, jnp.zeros((pad,), jnp.int32)])
    dstp = jnp.concatenate([dst, jnp.full((pad,), N, jnp.int32)])
    pk = (dstp * 16384 + srcp).reshape(NC, NS, CH, CW)
    batchr = batch.reshape(N // BLK, 1, BLK)

    z = _zkern(x, Wl0.T)
    agg, cntp = _sc_agg0(z, pk)
    inv = _prep_inv(cntp)
    h1 = _comb(agg, inv, x, Wr0.T, bl0)

    z = _zkern(h1, Wl1.T)
    (agg,) = _sc_agg(z, pk)
    h2 = _comb(agg, inv, h1, Wr1.T, bl1)

    z = _zkern(h2, Wl2.T)
    (agg,) = _sc_agg(z, pk)
    h3 = _comb(agg, inv, h2, Wr2.T, bl2)

    ps, gc = _pool(batchr, h1, h2, h3)
    return _mlp(ps, gc, Wfc1.T, bfc1, Wfc2.T, bfc2)


# speedup vs baseline: 5.1918x; 1.1225x over previous
import jax
import jax.numpy as jnp
from jax import lax
from jax.experimental import pallas as pl
from jax.experimental.pallas import tpu as pltpu
from jax.experimental.pallas import tpu_sc as plsc

N = 10000
E = 320000
D = 128
H = 128
G = 64
T = 10

NC = 2
NS = 16
CW = 128
CH = 79
EP = NC * NS * CH * CW
N_PAD = 10112
RPT = N_PAD // NS
BLK = 1000


def _make_sc_agg(do_cnt: bool):
    mesh = plsc.VectorSubcoreMesh(
        core_axis_name="c", subcore_axis_name="s", num_cores=NC,
        num_subcores=NS)
    out_type = [jax.ShapeDtypeStruct((NC, N_PAD, H), jnp.float32)]
    scratch = [
        pltpu.VMEM((CH, CW), jnp.int32),
        pltpu.VMEM((2, CW), jnp.int32),
        pltpu.VMEM((2, CW), jnp.int32),
        pltpu.VMEM((2, CW, H), jnp.float32),
        pltpu.VMEM_SHARED((N_PAD, H), jnp.float32),
        pltpu.SemaphoreType.DMA,
    ]
    if do_cnt:
        out_type.append(jax.ShapeDtypeStruct((NC, N_PAD, H), jnp.float32))

    def _fill(ref, value16):
        @pl.loop(0, CW)
        def _(r):
            for kk in range(H // 16):
                ref[r, pl.ds(kk * 16, 16)] = value16

    def _zero_acc_slice(rows_v, acc, s):
        for t in range(4):
            pltpu.sync_copy(rows_v,
                            acc.at[pl.ds(s * RPT + t * CW, CW)])
        pltpu.sync_copy(rows_v.at[pl.ds(0, RPT - 4 * CW)],
                        acc.at[pl.ds(s * RPT + 4 * CW, RPT - 4 * CW)])

    def body(z, pk, *rest):
        if do_cnt:
            agg, cntp, pk_v, src_v, dst_v, rows_v, acc, sem = rest
        else:
            agg, pk_v, src_v, dst_v, rows_v, acc, sem = rest
        c = lax.axis_index("c")
        s = lax.axis_index("s")

        pltpu.sync_copy(pk.at[c, s], pk_v)

        def _decode(j, b):
            for kk in range(CW // 16):
                p = pk_v[j, pl.ds(kk * 16, 16)]
                dst_v[b, pl.ds(kk * 16, 16)] = p >> 14
                src_v[b, pl.ds(kk * 16, 16)] = p & 16383

        def _issue_gather(j, b):
            _decode(j, b)
            pltpu.async_copy(z.at[src_v.at[b]], rows_v.at[b], sem)

        _fill(rows_v.at[0], jnp.zeros((16,), jnp.float32))
        _zero_acc_slice(rows_v.at[0], acc, s)
        plsc.subcore_barrier()

        _issue_gather(0, 0)

        @pl.loop(0, CH)
        def _(j):
            b = lax.rem(j, 2)
            pltpu.make_async_copy(z.at[src_v.at[b]], rows_v.at[b],
                                  sem).wait()

            @pl.when(j < CH - 1)
            def _():
                _issue_gather(j + 1, 1 - b)

            pltpu.sync_copy(rows_v.at[b], acc.at[dst_v.at[b]], add=True)

        plsc.subcore_barrier()
        pltpu.sync_copy(acc.at[pl.ds(s * RPT, RPT)],
                        agg.at[c, pl.ds(s * RPT, RPT)])

        if do_cnt:
            plsc.subcore_barrier()
            _fill(rows_v.at[0], jnp.zeros((16,), jnp.float32))
            _zero_acc_slice(rows_v.at[0], acc, s)
            _fill(rows_v.at[0], jnp.ones((16,), jnp.float32))
            plsc.subcore_barrier()

            @pl.loop(0, CH)
            def _(j):
                _decode(j, 0)
                pltpu.sync_copy(rows_v.at[0], acc.at[dst_v.at[0]],
                                add=True)

            plsc.subcore_barrier()
            pltpu.sync_copy(acc.at[pl.ds(s * RPT, RPT)],
                            cntp.at[c, pl.ds(s * RPT, RPT)])

    return pl.kernel(body, out_type=out_type, mesh=mesh,
                     scratch_types=scratch)


_sc_agg0 = _make_sc_agg(True)
_sc_agg = _make_sc_agg(False)


def _zkern(h, WlT):
    def body(h_ref, w_ref, o_ref):
        o_ref[...] = jnp.dot(h_ref[...], w_ref[...],
                             preferred_element_type=jnp.float32)

    return pl.pallas_call(
        body,
        grid=(N // BLK,),
        in_specs=[
            pl.BlockSpec((BLK, H), lambda i: (i, 0)),
            pl.BlockSpec((H, H), lambda i: (0, 0)),
        ],
        out_specs=pl.BlockSpec((BLK, H), lambda i: (i, 0)),
        out_shape=jax.ShapeDtypeStruct((N, H), jnp.float32),
    )(h, WlT)


def _prep_inv(cntp):
    def body(c_ref, o_ref):
        cnt = c_ref[0, :, 0:1] + c_ref[1, :, 0:1]
        o_ref[...] = 1.0 / jnp.maximum(cnt, 1.0)

    return pl.pallas_call(
        body,
        out_shape=jax.ShapeDtypeStruct((N_PAD, 1), jnp.float32),
    )(cntp)


def _comb(agg, inv, h, WrT, bl):
    def body(a0_ref, a1_ref, inv_ref, h_ref, wr_ref, b_ref, o_ref):
        asum = a0_ref[0] + a1_ref[0]
        o_ref[...] = (inv_ref[...] * asum + b_ref[...]
                      + jnp.dot(h_ref[...], wr_ref[...],
                                preferred_element_type=jnp.float32))

    return pl.pallas_call(
        body,
        grid=(N // BLK,),
        in_specs=[
            pl.BlockSpec((1, BLK, H), lambda i: (0, i, 0)),
            pl.BlockSpec((1, BLK, H), lambda i: (1, i, 0)),
            pl.BlockSpec((BLK, 1), lambda i: (i, 0)),
            pl.BlockSpec((BLK, H), lambda i: (i, 0)),
            pl.BlockSpec((H, H), lambda i: (0, 0)),
            pl.BlockSpec((1, H), lambda i: (0, 0)),
        ],
        out_specs=pl.BlockSpec((BLK, H), lambda i: (i, 0)),
        out_shape=jax.ShapeDtypeStruct((N, H), jnp.float32),
    )(agg, agg, inv, h, WrT, bl.reshape(1, -1))


def _pool(batchr, h1, h2, h3):
    def body(b_ref, h1_ref, h2_ref, h3_ref, ps_ref, gc_ref):
        i = pl.program_id(0)

        @pl.when(i == 0)
        def _():
            ps_ref[...] = jnp.zeros_like(ps_ref)
            gc_ref[...] = jnp.zeros_like(gc_ref)

        b = b_ref[0]
        S = (lax.broadcasted_iota(jnp.int32, (G, BLK), 0) == b
             ).astype(jnp.float32)
        cat = jnp.concatenate([h1_ref[...], h2_ref[...], h3_ref[...]],
                              axis=1)
        ps_ref[...] += jnp.dot(S, cat, preferred_element_type=jnp.float32)
        gc_ref[...] += jnp.sum(S, axis=1, keepdims=True)

    return pl.pallas_call(
        body,
        grid=(N // BLK,),
        in_specs=[
            pl.BlockSpec((1, 1, BLK), lambda i: (i, 0, 0)),
            pl.BlockSpec((BLK, H), lambda i: (i, 0)),
            pl.BlockSpec((BLK, H), lambda i: (i, 0)),
            pl.BlockSpec((BLK, H), lambda i: (i, 0)),
        ],
        out_specs=[
            pl.BlockSpec((G, 3 * H), lambda i: (0, 0)),
            pl.BlockSpec((G, 1), lambda i: (0, 0)),
        ],
        out_shape=[
            jax.ShapeDtypeStruct((G, 3 * H), jnp.float32),
            jax.ShapeDtypeStruct((G, 1), jnp.float32),
        ],
    )(batchr, h1, h2, h3)


def _mlp(ps, gc, W1T, b1, W2T, b2):
    def body(ps_ref, gc_ref, w1_ref, b1_ref, w2_ref, b2_ref, o_ref):
        pooled = ps_ref[...] / jnp.maximum(gc_ref[...], 1.0)
        hh = jax.nn.relu(jnp.dot(pooled, w1_ref[...],
                                 preferred_element_type=jnp.float32)
                         + b1_ref[...])
        o_ref[...] = jax.nn.sigmoid(
            jnp.dot(hh, w2_ref[...], preferred_element_type=jnp.float32)
            + b2_ref[...])

    return pl.pallas_call(
        body,
        out_shape=jax.ShapeDtypeStruct((G, T), jnp.float32),
    )(ps, gc, W1T, b1.reshape(1, -1), W2T, b2.reshape(1, -1))


def kernel(x, edge_index, batch, Wl0, bl0, Wr0, Wl1, bl1, Wr1, Wl2, bl2, Wr2,
           Wfc1, bfc1, Wfc2, bfc2):
    src = edge_index[0]
    dst = edge_index[1]
    pad = EP - E
    srcp = jnp.concatenate([src, jnp.zeros((pad,), jnp.int32)])
    dstp = jnp.concatenate([dst, jnp.full((pad,), N, jnp.int32)])
    pk = (dstp * 16384 + srcp).reshape(NC, NS, CH, CW)
    batchr = batch.reshape(N // BLK, 1, BLK)

    z = _zkern(x, Wl0.T)
    agg, cntp = _sc_agg0(z, pk)
    inv = _prep_inv(cntp)
    h1 = _comb(agg, inv, x, Wr0.T, bl0)

    z = _zkern(h1, Wl1.T)
    (agg,) = _sc_agg(z, pk)
    h2 = _comb(agg, inv, h1, Wr1.T, bl1)

    z = _zkern(h2, Wl2.T)
    (agg,) = _sc_agg(z, pk)
    h3 = _comb(agg, inv, h2, Wr2.T, bl2)

    ps, gc = _pool(batchr, h1, h2, h3)
    return _mlp(ps, gc, Wfc1.T, bfc1, Wfc2.T, bfc2)
